# Initial kernel scaffold; baseline (speedup 1.0000x reference)
import jax
import jax.numpy as jnp
from jax.experimental import pallas as pl


def kernel(bos, table):
    raise NotImplementedError("write your pallas kernel here")


# speedup vs baseline: 70.1940x; 70.1940x over previous
import functools

import jax
import jax.numpy as jnp
from jax import lax
from jax.experimental import pallas as pl
from jax.experimental.pallas import tpu as pltpu
from jax.experimental.pallas import tpu_sc as plsc

_VOCAB = 1000
_D = 32
_BATCH = 16384
_HIST = 200
_HIST_PAD = 208
_LANES = 16
_NC = 2
_NS = 16
_NW = _NC * _NS
_ROWS_PER_W = _BATCH // _NW
_CHUNK = 256

_GATHER_DNUMS = lax.GatherDimensionNumbers(
    offset_dims=(), collapsed_slice_dims=(0,), start_index_map=(0,))


def _lane_broadcast(vec, j):
  idx = jnp.full((_LANES, 1), j, dtype=jnp.int32)
  return lax.gather(vec, idx, _GATHER_DNUMS, (1,),
                    mode=lax.GatherScatterMode.PROMISE_IN_BOUNDS)


def _body(bos_hbm, table_hbm, out_hbm, bos_v, table_v, out_v):
  wid = lax.axis_index("s") * _NC + lax.axis_index("c")
  base = wid * _ROWS_PER_W

  pltpu.sync_copy(table_hbm, table_v)
  zeros16 = jnp.zeros((_LANES,), jnp.float32)
  table_v[pl.ds(0, _LANES)] = zeros16
  table_v[pl.ds(_LANES, _LANES)] = zeros16

  iota0 = lax.iota(jnp.int32, _LANES)
  iota1 = iota0 + _LANES

  def row_body(r, _):
    acc0 = jnp.zeros((_LANES,), jnp.float32)
    acc1 = jnp.zeros((_LANES,), jnp.float32)
    cntv = jnp.zeros((_LANES,), jnp.int32)
    for g in range(_HIST_PAD // _LANES):
      tokv = bos_v[r, pl.ds(g * _LANES, _LANES)]
      cntv = cntv + jnp.where(tokv != 0, 1, 0).astype(jnp.int32)
      tokd = tokv * _D
      for j in range(_LANES):
        b = _lane_broadcast(tokd, j)
        acc0 = acc0 + plsc.load_gather(table_v, [b + iota0])
        acc1 = acc1 + plsc.load_gather(table_v, [b + iota1])
    cntf = jnp.sum(cntv).astype(jnp.float32)
    out_v[r, pl.ds(0, _LANES)] = acc0 / cntf
    out_v[r, pl.ds(_LANES, _LANES)] = acc1 / cntf
    return 0

  for c in range(_ROWS_PER_W // _CHUNK):
    row0 = base + c * _CHUNK
    pltpu.sync_copy(bos_hbm.at[pl.ds(row0, _CHUNK)], bos_v)
    lax.fori_loop(0, _CHUNK, row_body, 0)
    pltpu.sync_copy(out_v, out_hbm.at[pl.ds(row0, _CHUNK)])


@functools.partial(jax.jit, donate_argnums=())
def _run(bos_p, table):
  mesh = plsc.VectorSubcoreMesh(core_axis_name="c", subcore_axis_name="s")
  k = pl.kernel(
      _body,
      out_type=jax.ShapeDtypeStruct((_BATCH, _D), jnp.float32),
      mesh=mesh,
      scratch_types=[
          pltpu.VMEM((_CHUNK, _HIST_PAD), jnp.int32),
          pltpu.VMEM((_VOCAB * _D,), jnp.float32),
          pltpu.VMEM((_CHUNK, _D), jnp.float32),
      ],
      compiler_params=pltpu.CompilerParams(needs_layout_passes=False),
  )
  return k(bos_p, table)


def kernel(bos, table):
  bos_p = jnp.pad(bos, ((0, 0), (0, _HIST_PAD - _HIST)))
  return _run(bos_p, table.reshape(-1))


# speedup vs baseline: 95.2735x; 1.3573x over previous
import functools

import jax
import jax.numpy as jnp
from jax import lax
from jax.experimental import pallas as pl
from jax.experimental.pallas import tpu as pltpu
from jax.experimental.pallas import tpu_sc as plsc

_VOCAB = 1000
_D = 32
_W = _D // 2
_BATCH = 16384
_HIST = 200
_HIST_PAD = 208
_LANES = 16
_NC = 2
_NS = 16
_NW = _NC * _NS
_ROWS_PER_W = _BATCH // _NW
_CHUNK = 256

_GATHER_DNUMS = lax.GatherDimensionNumbers(
    offset_dims=(), collapsed_slice_dims=(0,), start_index_map=(0,))


def _lane_broadcast(vec, j):
  idx = jnp.full((_LANES, 1), j, dtype=jnp.int32)
  return lax.gather(vec, idx, _GATHER_DNUMS, (1,),
                    mode=lax.GatherScatterMode.PROMISE_IN_BOUNDS)


def _body(bos_hbm, table_hbm, out_hbm, bos_v, table_v, out_v):
  wid = lax.axis_index("s") * _NC + lax.axis_index("c")
  base = wid * _ROWS_PER_W

  pltpu.sync_copy(table_hbm, table_v)
  table_v[pl.ds(0, _W)] = jnp.zeros((_W,), jnp.int32)

  iota = lax.iota(jnp.int32, _LANES)
  s_even = iota * 2
  s_odd = s_even + 1
  bf_zero = jnp.zeros((2 * _LANES,), jnp.bfloat16)

  def row_body(r, _):
    acc_e = jnp.zeros((_LANES,), jnp.float32)
    acc_o = jnp.zeros((_LANES,), jnp.float32)
    cntv = jnp.zeros((_LANES,), jnp.int32)
    for g in range(_HIST_PAD // _LANES):
      tokv = bos_v[r, pl.ds(g * _LANES, _LANES)]
      cntv = cntv + jnp.where(tokv != 0, 1, 0).astype(jnp.int32)
      tokw = tokv * _W
      acc_p = bf_zero
      for j in range(_LANES):
        b = _lane_broadcast(tokw, j)
        w = plsc.load_gather(table_v, [b + iota])
        acc_p = acc_p + plsc.bitcast(w, jnp.bfloat16)
      e, o = plsc.unpack(acc_p, format=plsc.PackFormat.INTERLEAVED)
      acc_e = acc_e + e
      acc_o = acc_o + o
    cntf = jnp.sum(cntv).astype(jnp.float32)
    base_r = r * _D
    plsc.store_scatter(out_v, [base_r + s_even], acc_e / cntf)
    plsc.store_scatter(out_v, [base_r + s_odd], acc_o / cntf)
    return 0

  for c in range(_ROWS_PER_W // _CHUNK):
    row0 = base + c * _CHUNK
    pltpu.sync_copy(bos_hbm.at[pl.ds(row0, _CHUNK)], bos_v)
    lax.fori_loop(0, _CHUNK, row_body, 0)
    pltpu.sync_copy(out_v, out_hbm.at[pl.ds(row0 * _D, _CHUNK * _D)])


@functools.partial(jax.jit, donate_argnums=())
def _run(bos_p, table_packed):
  mesh = plsc.VectorSubcoreMesh(core_axis_name="c", subcore_axis_name="s")
  k = pl.kernel(
      _body,
      out_type=jax.ShapeDtypeStruct((_BATCH * _D,), jnp.float32),
      mesh=mesh,
      scratch_types=[
          pltpu.VMEM((_CHUNK, _HIST_PAD), jnp.int32),
          pltpu.VMEM((_VOCAB * _W,), jnp.int32),
          pltpu.VMEM((_CHUNK * _D,), jnp.float32),
      ],
      compiler_params=pltpu.CompilerParams(needs_layout_passes=False),
  )
  return k(bos_p, table_packed)


def _pack_table(table):
  u16 = lax.bitcast_convert_type(table.astype(jnp.bfloat16), jnp.uint16)
  u32 = u16.astype(jnp.uint32).reshape(_VOCAB, _W, 2)
  words = u32[..., 0] | (u32[..., 1] << 16)
  return lax.bitcast_convert_type(words, jnp.int32).reshape(-1)


def kernel(bos, table):
  bos_p = jnp.pad(bos, ((0, 0), (0, _HIST_PAD - _HIST)))
  return _run(bos_p, _pack_table(table)).reshape(_BATCH, _D)
